# Initial kernel scaffold; baseline (speedup 1.0000x reference)
import jax
import jax.numpy as jnp
from jax.experimental import pallas as pl


def kernel(coords, angles, theta0, k):
    raise NotImplementedError("write your pallas kernel here")


# speedup vs baseline: 68.6989x; 68.6989x over previous
import functools

import jax
import jax.numpy as jnp
from jax import lax
from jax.experimental import pallas as pl
from jax.experimental.pallas import tpu as pltpu
from jax.experimental.pallas import tpu_sc as plsc

N = 100000
A = 3200000

NC = 2
NS = 16
NW = NC * NS
L = 16

T = A // NW
B = 4000
NBLK = T // B
NJ = B // L

_MAGIC = 0x5F3759DF

_ACOS = (1.5707963050, -0.2145988016, 0.0889789874, -0.0501743046,
         0.0308918810, -0.0170881256, 0.0066700901, -0.0012624911)


def _rsqrt(x, iters=3):
    i = plsc.bitcast(x, jnp.int32)
    y = plsc.bitcast(jnp.int32(_MAGIC) - (i >> 1), jnp.float32)
    for _ in range(iters):
        y = y * (1.5 - 0.5 * x * y * y)
    return y


def _acos(x):
    t = jnp.abs(x)
    u = 1.0 - t
    s = u * _rsqrt(jnp.maximum(u, 1e-30))
    p = jnp.full((L,), _ACOS[7], dtype=jnp.float32)
    for c in _ACOS[6::-1]:
        p = p * t + c
    r = s * p
    return jnp.where(x < 0, jnp.float32(jnp.pi) - r, r)


def _sc_body(xs_hbm, ys_hbm, zs_hbm, a0_hbm, ac_hbm, a2_hbm, th_hbm, kk_hbm,
             out_hbm,
             x_sh, y_sh, z_sh,
             idx0_v, idxc_v, idx2_v,
             x0_v, y0_v, z0_v, xc_v, yc_v, zc_v, x2_v, y2_v, z2_v,
             th_v, kk_v, acc_v, sem0, sem1, sem2):
    cid = lax.axis_index("c")
    sid = lax.axis_index("s")
    wid = sid * NC + cid

    @pl.when(sid == 0)
    def _():
        pltpu.sync_copy(xs_hbm, x_sh)
        pltpu.sync_copy(ys_hbm, y_sh)
        pltpu.sync_copy(zs_hbm, z_sh)
    plsc.subcore_barrier()

    base = wid * T

    def blk_body(b, acc):
        off = base + b * B
        pltpu.sync_copy(a0_hbm.at[pl.ds(off, B)], idx0_v)
        pltpu.sync_copy(ac_hbm.at[pl.ds(off, B)], idxc_v)
        pltpu.sync_copy(a2_hbm.at[pl.ds(off, B)], idx2_v)
        pltpu.sync_copy(th_hbm.at[pl.ds(off, B)], th_v)
        pltpu.sync_copy(kk_hbm.at[pl.ds(off, B)], kk_v)

        d = [
            pltpu.async_copy(x_sh.at[idx0_v], x0_v, sem0),
            pltpu.async_copy(y_sh.at[idx0_v], y0_v, sem0),
            pltpu.async_copy(z_sh.at[idx0_v], z0_v, sem0),
            pltpu.async_copy(x_sh.at[idxc_v], xc_v, sem1),
            pltpu.async_copy(y_sh.at[idxc_v], yc_v, sem1),
            pltpu.async_copy(z_sh.at[idxc_v], zc_v, sem1),
            pltpu.async_copy(x_sh.at[idx2_v], x2_v, sem2),
            pltpu.async_copy(y_sh.at[idx2_v], y2_v, sem2),
            pltpu.async_copy(z_sh.at[idx2_v], z2_v, sem2),
        ]
        for dd in d:
            dd.wait()

        def j_body(j, acc):
            sl = pl.ds(j * L, L)
            v1x = x0_v[sl] - xc_v[sl]
            v1y = y0_v[sl] - yc_v[sl]
            v1z = z0_v[sl] - zc_v[sl]
            v2x = x2_v[sl] - xc_v[sl]
            v2y = y2_v[sl] - yc_v[sl]
            v2z = z2_v[sl] - zc_v[sl]
            dot = v1x * v2x + v1y * v2y + v1z * v2z
            n1 = v1x * v1x + v1y * v1y + v1z * v1z
            n2 = v2x * v2x + v2y * v2y + v2z * v2z
            inv = _rsqrt(jnp.maximum(n1 * n2, 1e-30))
            cos = jnp.clip(dot * inv, -1.0, 1.0)
            theta = _acos(cos)
            dth = theta - th_v[sl]
            return acc + dth * dth * kk_v[sl] * 0.5

        return lax.fori_loop(0, NJ, j_body, acc)

    acc = lax.fori_loop(0, NBLK, blk_body, jnp.zeros((L,), jnp.float32))
    acc_v[...] = acc
    pltpu.sync_copy(acc_v, out_hbm.at[wid])


@jax.jit
def _sc_call(xs, ys, zs, a0, ac, a2, theta0, k):
    mesh = plsc.VectorSubcoreMesh(core_axis_name="c", subcore_axis_name="s")
    f = functools.partial(
        pl.kernel,
        out_type=jax.ShapeDtypeStruct((NW, L), jnp.float32),
        mesh=mesh,
        scratch_types=(
            [pltpu.VMEM_SHARED((N,), jnp.float32)] * 3
            + [pltpu.VMEM((B,), jnp.int32)] * 3
            + [pltpu.VMEM((B,), jnp.float32)] * 9
            + [pltpu.VMEM((B,), jnp.float32)] * 2
            + [pltpu.VMEM((L,), jnp.float32)]
            + [pltpu.SemaphoreType.DMA] * 3
        ),
        compiler_params=pltpu.CompilerParams(needs_layout_passes=False),
    )(_sc_body)
    return f(xs, ys, zs, a0, ac, a2, theta0, k)


def kernel(coords, angles, theta0, k):
    xs = coords[:, 0]
    ys = coords[:, 1]
    zs = coords[:, 2]
    a0 = angles[:, 0]
    ac = angles[:, 1]
    a2 = angles[:, 2]
    partials = _sc_call(xs, ys, zs, a0, ac, a2, theta0, k)
    return jnp.sum(partials)


# speedup vs baseline: 111.1626x; 1.6181x over previous
import functools

import jax
import jax.numpy as jnp
from jax import lax
from jax.experimental import pallas as pl
from jax.experimental.pallas import tpu as pltpu
from jax.experimental.pallas import tpu_sc as plsc

N = 100000
A = 3200000

NC = 2
NS = 16
NW = NC * NS
L = 16

T = A // NW
B = 400
NBLK = T // B
NJ = B // L

_MAGIC = 0x5F3759DF

_ACOS = (1.5707963050, -0.2145988016, 0.0889789874, -0.0501743046,
         0.0308918810, -0.0170881256, 0.0066700901, -0.0012624911)


def _rsqrt(x, iters=3):
    i = plsc.bitcast(x, jnp.int32)
    y = plsc.bitcast(jnp.int32(_MAGIC) - (i >> 1), jnp.float32)
    for _ in range(iters):
        y = y * (1.5 - 0.5 * x * y * y)
    return y


def _acos(x):
    t = jnp.abs(x)
    u = 1.0 - t
    s = u * _rsqrt(jnp.maximum(u, 1e-30))
    p = jnp.full((L,), _ACOS[7], dtype=jnp.float32)
    for c in _ACOS[6::-1]:
        p = p * t + c
    r = s * p
    return jnp.where(x < 0, jnp.float32(jnp.pi) - r, r)


def _sc_body(tab_hbm, a0_hbm, ac_hbm, a2_hbm, th_hbm, kk_hbm, out_hbm,
             tab_sh, idx_v, rows_v, thk_v, acc_v, semS, semG, semT):
    cid = lax.axis_index("c")
    sid = lax.axis_index("s")
    wid = sid * NC + cid

    @pl.when(sid == 0)
    def _():
        pltpu.sync_copy(tab_hbm, tab_sh)
    plsc.subcore_barrier()

    base = wid * T
    iota = lax.iota(jnp.int32, L)
    idx_hbms = (a0_hbm, ac_hbm, a2_hbm)

    def fire_S(b, par):
        off = base + b * B
        for e in range(3):
            pltpu.async_copy(idx_hbms[e].at[pl.ds(off, B)],
                             idx_v[par][e], semS[par])

    def wait_S(par):
        for e in range(3):
            pltpu.make_async_copy(idx_hbms[e].at[pl.ds(0, B)],
                                  idx_v[par][e], semS[par]).wait()

    def fire_G(b, par):
        off = base + b * B
        for e in range(3):
            pltpu.async_copy(tab_sh.at[idx_v[par][e]], rows_v[par][e],
                             semG[par])
        pltpu.async_copy(th_hbm.at[pl.ds(off, B)], thk_v[par][0], semT[par])
        pltpu.async_copy(kk_hbm.at[pl.ds(off, B)], thk_v[par][1], semT[par])

    def wait_G(par):
        for e in range(3):
            pltpu.make_async_copy(tab_sh.at[idx_v[par][e]], rows_v[par][e],
                                  semG[par]).wait()
        for w in range(2):
            pltpu.make_async_copy(th_hbm.at[pl.ds(0, B)], thk_v[par][w],
                                  semT[par]).wait()

    def compute(par, acc):
        r0, rc, r2 = rows_v[par]
        th_ref, kk_ref = thk_v[par]
        cvec = [jnp.full((L,), c, jnp.int32) for c in range(3)]

        def j_body(j, acc):
            rid = j * L + iota
            p0 = [plsc.load_gather(r0, [rid, cvec[c]]) for c in range(3)]
            pc = [plsc.load_gather(rc, [rid, cvec[c]]) for c in range(3)]
            p2 = [plsc.load_gather(r2, [rid, cvec[c]]) for c in range(3)]
            v1x, v1y, v1z = (p0[0] - pc[0], p0[1] - pc[1], p0[2] - pc[2])
            v2x, v2y, v2z = (p2[0] - pc[0], p2[1] - pc[1], p2[2] - pc[2])
            dot = v1x * v2x + v1y * v2y + v1z * v2z
            n1 = v1x * v1x + v1y * v1y + v1z * v1z
            n2 = v2x * v2x + v2y * v2y + v2z * v2z
            inv = _rsqrt(jnp.maximum(n1 * n2, 1e-30))
            cos = jnp.clip(dot * inv, -1.0, 1.0)
            theta = _acos(cos)
            sl = pl.ds(j * L, L)
            dth = theta - th_ref[sl]
            return acc + dth * dth * kk_ref[sl] * 0.5

        return lax.fori_loop(0, NJ, j_body, acc)

    fire_S(0, 0)
    fire_S(1, 1)
    wait_S(0)
    fire_G(0, 0)

    def pair_body(p, acc):
        b = 2 * p
        wait_S(1)
        fire_G(b + 1, 1)
        wait_G(0)

        @pl.when(b + 2 < NBLK)
        def _():
            fire_S(b + 2, 0)
        acc = compute(0, acc)

        @pl.when(b + 2 < NBLK)
        def _():
            wait_S(0)
            fire_G(b + 2, 0)
        wait_G(1)

        @pl.when(b + 3 < NBLK)
        def _():
            fire_S(b + 3, 1)
        acc = compute(1, acc)
        return acc

    acc = lax.fori_loop(0, NBLK // 2, pair_body, jnp.zeros((L,), jnp.float32))
    acc_v[...] = acc
    pltpu.sync_copy(acc_v, out_hbm.at[wid])


@jax.jit
def _sc_call(tab, a0, ac, a2, theta0, k):
    mesh = plsc.VectorSubcoreMesh(core_axis_name="c", subcore_axis_name="s")
    f = functools.partial(
        pl.kernel,
        out_type=jax.ShapeDtypeStruct((NW, L), jnp.float32),
        mesh=mesh,
        scratch_types=(
            [pltpu.VMEM_SHARED((N, 8), jnp.float32)]
            + [[[pltpu.VMEM((B,), jnp.int32)] * 3] * 2]
            + [[[pltpu.VMEM((B, 8), jnp.float32)] * 3] * 2]
            + [[[pltpu.VMEM((B,), jnp.float32)] * 2] * 2]
            + [pltpu.VMEM((L,), jnp.float32)]
            + [[pltpu.SemaphoreType.DMA] * 2] * 3
        ),
        compiler_params=pltpu.CompilerParams(
            needs_layout_passes=False, use_tc_tiling_on_sc=False),
    )(_sc_body)
    return f(tab, a0, ac, a2, theta0, k)


def kernel(coords, angles, theta0, k):
    tab = jnp.concatenate(
        [coords, jnp.zeros((N, 5), jnp.float32)], axis=1)
    a0 = angles[:, 0]
    ac = angles[:, 1]
    a2 = angles[:, 2]
    partials = _sc_call(tab, a0, ac, a2, theta0, k)
    return jnp.sum(partials)
